# baseline (speedup 1.0000x reference)
import jax
import jax.numpy as jnp
from jax.experimental import pallas as pl
from jax.experimental.pallas import tpu as pltpu


def _addend_body(months_ref, ch_ref, pos_ref, mon_ref, out_ref):
    b, t, bs, d = out_ref.shape
    n = ch_ref.shape[1]
    ch = ch_ref[...]
    zero = jnp.zeros((bs, n), jnp.float32)
    for bi in range(b):
        for ti in range(t):
            m = months_ref[bi, ti]
            row_m = mon_ref[pl.ds(m, 1), :]
            row3 = jnp.concatenate([
                ch,
                jnp.broadcast_to(pos_ref[ti:ti + 1, :], (bs, n)),
                jnp.broadcast_to(row_m, (bs, n)),
                zero,
            ], axis=-1)
            out_ref[bi, ti] = row3


def _add_body(tok_ref, a_ref, out_ref):
    a = a_ref[...]
    out_ref[...] = tok_ref[...] + a[:, None, :, :, None, :]


def kernel(tokens, timestamps, channel_embed, pos_embed, month_table):
    b, h, w, t, bs, d = tokens.shape
    months = timestamps[:, :, 1].astype(jnp.int32)

    a_small = pl.pallas_call(
        _addend_body,
        in_specs=[
            pl.BlockSpec(memory_space=pltpu.SMEM),
            pl.BlockSpec(memory_space=pltpu.VMEM),
            pl.BlockSpec(memory_space=pltpu.VMEM),
            pl.BlockSpec(memory_space=pltpu.VMEM),
        ],
        out_shape=jax.ShapeDtypeStruct((b, t, bs, d), jnp.float32),
    )(months, channel_embed, pos_embed, month_table)

    tok_t = jnp.transpose(tokens, (0, 1, 3, 4, 2, 5))

    H = 1
    out_t = pl.pallas_call(
        _add_body,
        grid=(b, h // H),
        in_specs=[
            pl.BlockSpec((1, H, t, bs, w, d), lambda i, j: (i, j, 0, 0, 0, 0)),
            pl.BlockSpec((1, t, bs, d), lambda i, j: (i, 0, 0, 0)),
        ],
        out_specs=pl.BlockSpec((1, H, t, bs, w, d), lambda i, j: (i, j, 0, 0, 0, 0)),
        out_shape=jax.ShapeDtypeStruct(tok_t.shape, tokens.dtype),
    )(tok_t, a_small)
    return jnp.transpose(out_t, (0, 1, 4, 2, 3, 5))


# speedup vs baseline: 1.1840x; 1.1840x over previous
import jax
import jax.numpy as jnp
from jax.experimental import pallas as pl
from jax.experimental.pallas import tpu as pltpu


def _addend_body(months_ref, ch_ref, pos_ref, mon_ref, out_ref):
    b, t, bs, d = out_ref.shape
    n = ch_ref.shape[1]
    ch = ch_ref[...]
    zero = jnp.zeros((bs, n), jnp.float32)
    for bi in range(b):
        for ti in range(t):
            m = months_ref[bi, ti]
            row_m = mon_ref[pl.ds(m, 1), :]
            row3 = jnp.concatenate([
                ch,
                jnp.broadcast_to(pos_ref[ti:ti + 1, :], (bs, n)),
                jnp.broadcast_to(row_m, (bs, n)),
                zero,
            ], axis=-1)
            out_ref[bi, ti] = row3


def _add_body(tok_ref, a_ref, out_ref):
    a = a_ref[...]
    out_ref[...] = tok_ref[...] + a[:, None, :, :, None, :]


def kernel(tokens, timestamps, channel_embed, pos_embed, month_table):
    b, h, w, t, bs, d = tokens.shape
    months = timestamps[:, :, 1].astype(jnp.int32)

    a_small = pl.pallas_call(
        _addend_body,
        in_specs=[
            pl.BlockSpec(memory_space=pltpu.SMEM),
            pl.BlockSpec(memory_space=pltpu.VMEM),
            pl.BlockSpec(memory_space=pltpu.VMEM),
            pl.BlockSpec(memory_space=pltpu.VMEM),
        ],
        out_shape=jax.ShapeDtypeStruct((b, t, bs, d), jnp.float32),
    )(months, channel_embed, pos_embed, month_table)

    tok_t = jnp.transpose(tokens, (0, 1, 3, 4, 2, 5))

    H = 4
    out_t = pl.pallas_call(
        _add_body,
        grid=(b, h // H),
        in_specs=[
            pl.BlockSpec((1, H, t, bs, w, d), lambda i, j: (i, j, 0, 0, 0, 0)),
            pl.BlockSpec((1, t, bs, d), lambda i, j: (i, 0, 0, 0)),
        ],
        out_specs=pl.BlockSpec((1, H, t, bs, w, d), lambda i, j: (i, j, 0, 0, 0, 0)),
        out_shape=jax.ShapeDtypeStruct(tok_t.shape, tokens.dtype),
    )(tok_t, a_small)
    return jnp.transpose(out_t, (0, 1, 4, 2, 3, 5))


# speedup vs baseline: 1.1984x; 1.0121x over previous
import jax
import jax.numpy as jnp
from jax.experimental import pallas as pl
from jax.experimental.pallas import tpu as pltpu


def _addend_body(months_ref, ch_ref, pos_ref, mon_ref, out_ref):
    b, t, bs, d = out_ref.shape
    n = ch_ref.shape[1]
    ch = ch_ref[...]
    zero = jnp.zeros((bs, n), jnp.float32)
    for bi in range(b):
        for ti in range(t):
            m = months_ref[bi, ti]
            row_m = mon_ref[pl.ds(m, 1), :]
            row3 = jnp.concatenate([
                ch,
                jnp.broadcast_to(pos_ref[ti:ti + 1, :], (bs, n)),
                jnp.broadcast_to(row_m, (bs, n)),
                zero,
            ], axis=-1)
            out_ref[bi, ti] = row3


def _add_body(tok_ref, a_ref, out_ref):
    a = a_ref[...]
    out_ref[...] = tok_ref[...] + a[:, None, :, :, None, :]


def kernel(tokens, timestamps, channel_embed, pos_embed, month_table):
    b, h, w, t, bs, d = tokens.shape
    months = timestamps[:, :, 1].astype(jnp.int32)

    a_small = pl.pallas_call(
        _addend_body,
        in_specs=[
            pl.BlockSpec(memory_space=pltpu.SMEM),
            pl.BlockSpec(memory_space=pltpu.VMEM),
            pl.BlockSpec(memory_space=pltpu.VMEM),
            pl.BlockSpec(memory_space=pltpu.VMEM),
        ],
        out_shape=jax.ShapeDtypeStruct((b, t, bs, d), jnp.float32),
    )(months, channel_embed, pos_embed, month_table)

    tok_t = jnp.transpose(tokens, (0, 1, 3, 4, 2, 5))

    H = 8
    out_t = pl.pallas_call(
        _add_body,
        grid=(b, h // H),
        compiler_params=pltpu.CompilerParams(vmem_limit_bytes=120 * 1024 * 1024),
        in_specs=[
            pl.BlockSpec((1, H, t, bs, w, d), lambda i, j: (i, j, 0, 0, 0, 0)),
            pl.BlockSpec((1, t, bs, d), lambda i, j: (i, 0, 0, 0)),
        ],
        out_specs=pl.BlockSpec((1, H, t, bs, w, d), lambda i, j: (i, j, 0, 0, 0, 0)),
        out_shape=jax.ShapeDtypeStruct(tok_t.shape, tokens.dtype),
    )(tok_t, a_small)
    return jnp.transpose(out_t, (0, 1, 4, 2, 3, 5))


# speedup vs baseline: 1.2128x; 1.0120x over previous
import jax
import jax.numpy as jnp
from jax.experimental import pallas as pl
from jax.experimental.pallas import tpu as pltpu


def _fused_body(ts_ref, tok_ref, ch_ref, pos_ref, mon_ref, out_ref, a_ref):
    i = pl.program_id(0)
    j = pl.program_id(1)

    @pl.when(jnp.logical_and(i == 0, j == 0))
    def _():
        bq, tq, bsq, _ = a_ref.shape
        n = ch_ref.shape[1]
        ch = ch_ref[...]
        zero = jnp.zeros((bsq, n), jnp.float32)
        for bi in range(bq):
            for ti in range(tq):
                m = ts_ref[bi, ti, 1]
                row_m = mon_ref[pl.ds(m, 1), :]
                a_ref[bi, ti] = jnp.concatenate([
                    ch,
                    jnp.broadcast_to(pos_ref[ti:ti + 1, :], (bsq, n)),
                    jnp.broadcast_to(row_m, (bsq, n)),
                    zero,
                ], axis=-1)

    a = a_ref[i]
    out_ref[...] = tok_ref[...] + a[None, None, :, :, None, :]


def kernel(tokens, timestamps, channel_embed, pos_embed, month_table):
    b, h, w, t, bs, d = tokens.shape

    tok_t = jnp.transpose(tokens, (0, 1, 3, 4, 2, 5))

    H = 8
    grid_spec = pltpu.PrefetchScalarGridSpec(
        num_scalar_prefetch=1,
        grid=(b, h // H),
        in_specs=[
            pl.BlockSpec((1, H, t, bs, w, d), lambda i, j, ts: (i, j, 0, 0, 0, 0)),
            pl.BlockSpec(channel_embed.shape, lambda i, j, ts: (0, 0)),
            pl.BlockSpec(pos_embed.shape, lambda i, j, ts: (0, 0)),
            pl.BlockSpec(month_table.shape, lambda i, j, ts: (0, 0)),
        ],
        out_specs=pl.BlockSpec((1, H, t, bs, w, d), lambda i, j, ts: (i, j, 0, 0, 0, 0)),
        scratch_shapes=[pltpu.VMEM((b, t, bs, d), jnp.float32)],
    )

    out_t = pl.pallas_call(
        _fused_body,
        grid_spec=grid_spec,
        compiler_params=pltpu.CompilerParams(vmem_limit_bytes=120 * 1024 * 1024),
        out_shape=jax.ShapeDtypeStruct(tok_t.shape, tokens.dtype),
    )(timestamps.astype(jnp.int32), tok_t, channel_embed, pos_embed, month_table)
    return jnp.transpose(out_t, (0, 1, 4, 2, 3, 5))
